# baseline (speedup 1.0000x reference)
import jax
import jax.numpy as jnp
from jax.experimental import pallas as pl
from jax.experimental.pallas import tpu as pltpu

D_MODEL = 128
SELECT_N = 2048


def _score_body(b_ref, x_ref, w_ref, s_ref):
    s = jax.lax.dot_general(
        w_ref[...], x_ref[0],
        (((1,), (0,)), ((), ())),
        preferred_element_type=jnp.float32,
    )
    s_ref[0] = s + b_ref[0]


def _scores(x, W, b):
    B, C, D, P = x.shape
    xf = x.reshape(B * C, D, P)
    return pl.pallas_call(
        _score_body,
        grid=(B * C,),
        in_specs=[
            pl.BlockSpec(memory_space=pltpu.SMEM),
            pl.BlockSpec((1, D, P), lambda i: (i, 0, 0)),
            pl.BlockSpec((1, D), lambda i: (0, 0)),
        ],
        out_specs=pl.BlockSpec((1, 1, P), lambda i: (i, 0, 0)),
        out_shape=jax.ShapeDtypeStruct((B * C, 1, P), jnp.float32),
    )(b, xf, W)


def kernel(x, W, b):
    B, C, D, P = x.shape
    scores = _scores(x, W, b).reshape(B, C * P)
    _, top_idx = jax.lax.top_k(scores, SELECT_N)
    patches = jnp.transpose(x, (0, 1, 3, 2)).reshape(B, C * P, D)
    selected = jnp.take_along_axis(patches, top_idx[:, :, None], axis=1)
    return selected
